# baseline (speedup 1.0000x reference)
import functools

import jax
import jax.numpy as jnp
from jax import lax
from jax.experimental import pallas as pl
from jax.experimental.pallas import tpu as pltpu
from jax.experimental.pallas import tpu_sc as plsc

_NC, _NS = 2, 16
_NW = _NC * _NS
_D = 64
_V = 1000000
_B = 4096
_S = 200
_RB = 256
_NBLK = _V // _RB
_TAIL = _V - _NBLK * _RB
_LAPS = 124


def _iota16():
    return lax.iota(jnp.int32, 16)


@functools.lru_cache(maxsize=None)
def _build_gather():
    b_per_w = _B // _NW
    mesh = plsc.VectorSubcoreMesh(
        core_axis_name="c", subcore_axis_name="s",
        num_cores=_NC, num_subcores=_NS)

    @functools.partial(
        pl.kernel,
        out_type=jax.ShapeDtypeStruct((_S, 8, _NW, 8, 128), jnp.float32),
        mesh=mesh,
        compiler_params=pltpu.CompilerParams(
            use_tc_tiling_on_sc=False, needs_layout_passes=False),
        scratch_types=[
            pltpu.VMEM((b_per_w, _S), jnp.int32),
            pltpu.VMEM((_S, b_per_w), jnp.int32),
            pltpu.VMEM((2, b_per_w, 128), jnp.float32),
            pltpu.VMEM((2, 8, 8, b_per_w + 8), jnp.float32),
            pltpu.SemaphoreType.DMA,
            pltpu.SemaphoreType.DMA,
            pltpu.SemaphoreType.DMA,
            pltpu.SemaphoreType.DMA,
        ],
    )
    def gkern(ids_hbm, tab_hbm, out_hbm, idx_v, idxT_v, rows_v, rT_v,
              gs0, gs1, ps0, ps1):
        wid = lax.axis_index("s") * _NC + lax.axis_index("c")
        b0 = wid * b_per_w
        gsems = (gs0, gs1)
        psems = (ps0, ps1)
        it = _iota16()

        pltpu.sync_copy(ids_hbm.at[pl.ds(b0, b_per_w)], idx_v)

        @plsc.parallel_loop(0, _S, 1, unroll=4)
        def _(s):
            sv = jnp.full((16,), s, jnp.int32)
            for bg in range(b_per_w // 16):
                v = plsc.load_gather(idx_v, [bg * 16 + it, sv])
                idxT_v[s, pl.ds(bg * 16, 16)] = v

        def g_start(s, buf):
            pltpu.async_copy(
                tab_hbm.at[idxT_v.at[s]], rows_v.at[buf], gsems[buf])

        def g_wait(buf):
            pltpu.make_async_copy(
                tab_hbm.at[pl.ds(0, b_per_w)], rows_v.at[buf], gsems[buf]
            ).wait()

        def put(s, buf):
            pltpu.async_copy(
                rT_v.at[buf, :, :, pl.ds(0, b_per_w)],
                out_hbm.at[s, :, wid], psems[buf])

        def p_wait(buf):
            pltpu.make_async_copy(
                rT_v.at[buf, :, :, pl.ds(0, b_per_w)],
                out_hbm.at[0, :, 0], psems[buf]).wait()

        def transpose_rows(buf):
            rows = rows_v.at[buf]
            rT = rT_v.at[buf]
            chi = [(cg * 16 + it) // 8 for cg in range(4)]
            clo = [(cg * 16 + it) % 8 for cg in range(4)]

            @plsc.parallel_loop(0, b_per_w, 1, unroll=8)
            def _(j):
                jv = jnp.full((16,), j, jnp.int32)
                for cg in range(4):
                    v = plsc.load_gather(
                        rows, [jv, cg * 16 + it])
                    plsc.store_scatter(rT, [chi[cg], clo[cg], jv], v)

        g_start(0, 0)
        g_start(1, 1)
        g_wait(0)
        transpose_rows(0)
        put(0, 0)
        g_start(2, 0)
        g_wait(1)
        transpose_rows(1)
        put(1, 1)
        g_start(3, 1)

        def body(t, carry):
            s = 2 + 2 * t
            g_wait(0)
            p_wait(0)
            transpose_rows(0)
            put(s, 0)

            @pl.when(s + 2 < _S)
            def _():
                g_start(s + 2, 0)

            g_wait(1)
            p_wait(1)
            transpose_rows(1)
            put(s + 1, 1)

            @pl.when(s + 3 < _S)
            def _():
                g_start(s + 3, 1)
            return carry

        lax.fori_loop(0, (_S - 2) // 2, body, 0)
        p_wait(0)
        p_wait(1)

    return gkern


def kernel(input_ids, table):
    tab = jnp.pad(table, ((0, 0), (0, 128 - _D)))
    out5d = _build_gather()(input_ids, tab)
    return out5d.transpose(2, 4, 0, 1, 3).reshape(_B, _S, _D)
